# baseline (speedup 1.0000x reference)
import functools

import jax
import jax.numpy as jnp
from jax import lax
from jax.experimental import pallas as pl
from jax.experimental.pallas import tpu as pltpu
from jax.experimental.pallas import tpu_sc as plsc

N_SRC = 4
N_TGT = 256
NV = 100
NL = 47
NLP = 48
TVP = 112
NKP = 64
I32MAX = 2**31 - 1


def _vperm(x, idx16):
    return lax.gather(
        x, idx16[:, None],
        lax.GatherDimensionNumbers(offset_dims=(), collapsed_slice_dims=(0,),
                                   start_index_map=(0,)),
        (1,), mode=lax.GatherScatterMode.PROMISE_IN_BOUNDS)


def _pack_views(A, F, C, E):
    n = A.shape[0]
    return jnp.concatenate(
        [A.reshape(n, NV, 35), F.reshape(n, NV, 10),
         C.reshape(n, NV, 1), E.reshape(n, NV, 1),
         jnp.zeros((n, NV, 1), jnp.int32)], axis=-1)


def _lfd_sc(q8, src, tgt, align):
    info = plsc.get_sparse_core_info()
    nw = info.num_cores * info.num_subcores
    jpw = N_TGT // nw
    mesh = plsc.VectorSubcoreMesh(core_axis_name="c", subcore_axis_name="s")

    @functools.partial(
        pl.kernel,
        out_type=jax.ShapeDtypeStruct((nw, N_SRC, jpw, 16), jnp.int32),
        mesh=mesh,
        compiler_params=pltpu.CompilerParams(use_tc_tiling_on_sc=False,
                                             needs_layout_passes=False),
        scratch_types=[
            pltpu.VMEM((65536,), jnp.float32),
            pltpu.VMEM((N_SRC, NV, NLP), jnp.int32),
            pltpu.VMEM((NLP, TVP), jnp.int32),
            pltpu.VMEM((10, NKP), jnp.int32),
            pltpu.VMEM((NV * TVP,), jnp.int32),
            pltpu.VMEM((NV * TVP,), jnp.float32),
            pltpu.VMEM((N_SRC, jpw, 16), jnp.int32),
        ],
    )
    def k(q8_hbm, src_hbm, tgt_hbm, align_hbm, out_hbm,
          q_v, src_v, tgt_v, align_v, cost_v, part_v, res_v):
        wid = lax.axis_index("s") * info.num_cores + lax.axis_index("c")
        pltpu.sync_copy(q8_hbm, q_v)
        pltpu.sync_copy(src_hbm, src_v)
        pltpu.sync_copy(align_hbm, align_v)

        lane_sel = [jnp.full((16,), m, jnp.int32) for m in range(16)]

        def per_j(jloc, _):
            pltpu.sync_copy(tgt_hbm.at[wid * jpw + jloc], tgt_v)

            def per_i(i, _):
                def per_tb(tb, _):
                    col = pl.ds(tb * 16, 16)
                    for c in range(3):
                        ls = range(16 * c, min(16 * c + 16, NL))
                        t_vec = {l: tgt_v[l, col] for l in ls}

                        def per_sv(sv, _, c=c, ls=ls, t_vec=t_vec):
                            sa = src_v[i, sv, pl.ds(c * 16, 16)]
                            a1 = [jnp.zeros((16,), jnp.float32)
                                  for _ in range(2)]
                            a2 = [jnp.zeros((16,), jnp.float32)
                                  for _ in range(2)]
                            n1 = n2 = 0
                            for l in ls:
                                row = _vperm(sa, lane_sel[l % 16])
                                g = plsc.load_gather(q_v, [row + t_vec[l]])
                                if 35 <= l <= 45:
                                    a2[n2 % 2] = a2[n2 % 2] + g
                                    n2 += 1
                                else:
                                    a1[n1 % 2] = a1[n1 % 2] + g
                                    n1 += 1
                            w = (a1[0] + a1[1]) + 2.0 * (a2[0] + a2[1])
                            sl = pl.ds(sv * TVP + tb * 16, 16)
                            if c == 0:
                                part_v[sl] = w
                            elif c == 1:
                                part_v[sl] = part_v[sl] + w
                            else:
                                cost_v[sl] = (
                                    part_v[sl] + w).astype(jnp.int32)
                            return 0

                        lax.fori_loop(0, NV, per_sv, 0)
                    return 0

                lax.fori_loop(0, TVP // 16, per_tb, 0)

                minv = jnp.full((16,), I32MAX, jnp.int32)
                for kb in range(NKP // 16):
                    a_vec = [align_v[dd, pl.ds(kb * 16, 16)]
                             for dd in range(10)]

                    def per_st(st, mv):
                        s1 = st // 10
                        t1 = st % 10
                        pa = [jnp.zeros((16,), jnp.int32) for _ in range(2)]
                        for dd in range(10):
                            base = jnp.full(
                                (16,), (s1 * 10 + dd) * TVP + t1 * 10,
                                jnp.int32)
                            pa[dd % 2] = pa[dd % 2] + plsc.load_gather(
                                cost_v, [base + a_vec[dd]])
                        return jnp.minimum(mv, pa[0] + pa[1])

                    minv = lax.fori_loop(0, NV, per_st, minv)

                m = lax.reduce_min(minv, (0,))
                res_v[i, jloc, :] = jnp.full((16,), m, jnp.int32)
                return 0

            lax.fori_loop(0, N_SRC, per_i, 0)
            return 0

        lax.fori_loop(0, jpw, per_j, 0)
        pltpu.sync_copy(res_v, out_hbm.at[wid])

    return k(q8, src, tgt, align)


def kernel(q8_table, align_10, src_ArtCoeff, src_FdCoeff_q8, src_CirCoeff_q8,
           src_EccCoeff_q8, tgt_ArtCoeff, tgt_FdCoeff_q8, tgt_CirCoeff_q8,
           tgt_EccCoeff_q8):
    src = _pack_views(src_ArtCoeff, src_FdCoeff_q8,
                      src_CirCoeff_q8, src_EccCoeff_q8) * 256
    tgtp = _pack_views(tgt_ArtCoeff, tgt_FdCoeff_q8,
                       tgt_CirCoeff_q8, tgt_EccCoeff_q8)
    tgt = jnp.zeros((N_TGT, NLP, TVP), jnp.int32)
    tgt = tgt.at[:, :, :NV].set(jnp.transpose(tgtp, (0, 2, 1)))
    align = jnp.concatenate(
        [align_10[:, :10].T,
         jnp.broadcast_to(align_10[0, :10][:, None], (10, NKP - 60))],
        axis=1).astype(jnp.int32)

    out = _lfd_sc(q8_table.reshape(-1), src, tgt, align)
    return jnp.transpose(out[:, :, :, 0], (1, 0, 2)).reshape(N_SRC, N_TGT)


# speedup vs baseline: 1.7939x; 1.7939x over previous
import functools

import jax
import jax.numpy as jnp
from jax import lax
from jax.experimental import pallas as pl
from jax.experimental.pallas import tpu as pltpu
from jax.experimental.pallas import tpu_sc as plsc

N_SRC = 4
N_TGT = 256
NV = 100
NL = 47
NLP = 48
TVP = 112
NKP = 64
I32MAX = 2**31 - 1


def _vperm(x, idx16):
    return lax.gather(
        x, idx16[:, None],
        lax.GatherDimensionNumbers(offset_dims=(), collapsed_slice_dims=(0,),
                                   start_index_map=(0,)),
        (1,), mode=lax.GatherScatterMode.PROMISE_IN_BOUNDS)


def _pack_views(A, F, C, E):
    n = A.shape[0]
    return jnp.concatenate(
        [A.reshape(n, NV, 35), F.reshape(n, NV, 10),
         C.reshape(n, NV, 1), E.reshape(n, NV, 1),
         jnp.zeros((n, NV, 1), jnp.int32)], axis=-1)


def _lfd_sc(q8, src, tgt, align):
    info = plsc.get_sparse_core_info()
    nw = info.num_cores * info.num_subcores
    jpw = N_TGT // nw
    mesh = plsc.VectorSubcoreMesh(core_axis_name="c", subcore_axis_name="s")

    @functools.partial(
        pl.kernel,
        out_type=jax.ShapeDtypeStruct((nw, N_SRC, jpw, 16), jnp.int32),
        mesh=mesh,
        compiler_params=pltpu.CompilerParams(use_tc_tiling_on_sc=False,
                                             needs_layout_passes=False),
        scratch_types=[
            pltpu.VMEM((65536,), jnp.float32),
            pltpu.VMEM((N_SRC, NV, NLP), jnp.int32),
            pltpu.VMEM((NLP, TVP), jnp.int32),
            pltpu.VMEM((10, NKP), jnp.int32),
            pltpu.VMEM((NV * TVP,), jnp.int32),
            pltpu.VMEM((NV * TVP,), jnp.float32),
            pltpu.VMEM((N_SRC, jpw, 16), jnp.int32),
        ],
    )
    def k(q8_hbm, src_hbm, tgt_hbm, align_hbm, out_hbm,
          q_v, src_v, tgt_v, align_v, cost_v, part_v, res_v):
        wid = lax.axis_index("s") * info.num_cores + lax.axis_index("c")
        pltpu.sync_copy(q8_hbm, q_v)
        pltpu.sync_copy(src_hbm, src_v)
        pltpu.sync_copy(align_hbm, align_v)

        lane_sel = [jnp.full((16,), m, jnp.int32) for m in range(16)]

        def per_j(jloc, _):
            pltpu.sync_copy(tgt_hbm.at[wid * jpw + jloc], tgt_v)

            def per_i(i, _):
                def per_tb(tb, _):
                    col = pl.ds(tb * 16, 16)
                    for c in range(3):
                        ls = range(16 * c, min(16 * c + 16, NL))
                        t_vec = {l: tgt_v[l, col] for l in ls}

                        @plsc.parallel_loop(0, NV, 1, unroll=2)
                        def per_sv(sv, c=c, ls=ls, t_vec=t_vec):
                            sa = src_v[i, sv, pl.ds(c * 16, 16)]
                            a1 = [jnp.zeros((16,), jnp.float32)
                                  for _ in range(2)]
                            a2 = [jnp.zeros((16,), jnp.float32)
                                  for _ in range(2)]
                            n1 = n2 = 0
                            for l in ls:
                                row = _vperm(sa, lane_sel[l % 16])
                                g = plsc.load_gather(q_v, [row + t_vec[l]])
                                if 35 <= l <= 45:
                                    a2[n2 % 2] = a2[n2 % 2] + g
                                    n2 += 1
                                else:
                                    a1[n1 % 2] = a1[n1 % 2] + g
                                    n1 += 1
                            w = (a1[0] + a1[1]) + 2.0 * (a2[0] + a2[1])
                            sl = pl.ds(sv * TVP + tb * 16, 16)
                            if c == 0:
                                part_v[sl] = w
                            elif c == 1:
                                part_v[sl] = part_v[sl] + w
                            else:
                                cost_v[sl] = (
                                    part_v[sl] + w).astype(jnp.int32)
                    return 0

                lax.fori_loop(0, TVP // 16, per_tb, 0)

                minv = jnp.full((16,), I32MAX, jnp.int32)
                for kb in range(NKP // 16):
                    a_vec = [align_v[dd, pl.ds(kb * 16, 16)]
                             for dd in range(10)]

                    @plsc.parallel_loop(0, NV, 1, unroll=2, carry=minv)
                    def per_st(st, mv):
                        s1 = st // 10
                        t1 = st % 10
                        pa = [jnp.zeros((16,), jnp.int32) for _ in range(2)]
                        for dd in range(10):
                            base = jnp.full(
                                (16,), (s1 * 10 + dd) * TVP + t1 * 10,
                                jnp.int32)
                            pa[dd % 2] = pa[dd % 2] + plsc.load_gather(
                                cost_v, [base + a_vec[dd]])
                        return jnp.minimum(mv, pa[0] + pa[1])

                    minv = per_st

                m = lax.reduce_min(minv, (0,))
                res_v[i, jloc, :] = jnp.full((16,), m, jnp.int32)
                return 0

            lax.fori_loop(0, N_SRC, per_i, 0)
            return 0

        lax.fori_loop(0, jpw, per_j, 0)
        pltpu.sync_copy(res_v, out_hbm.at[wid])

    return k(q8, src, tgt, align)


def kernel(q8_table, align_10, src_ArtCoeff, src_FdCoeff_q8, src_CirCoeff_q8,
           src_EccCoeff_q8, tgt_ArtCoeff, tgt_FdCoeff_q8, tgt_CirCoeff_q8,
           tgt_EccCoeff_q8):
    src = _pack_views(src_ArtCoeff, src_FdCoeff_q8,
                      src_CirCoeff_q8, src_EccCoeff_q8) * 256
    tgtp = _pack_views(tgt_ArtCoeff, tgt_FdCoeff_q8,
                       tgt_CirCoeff_q8, tgt_EccCoeff_q8)
    tgt = jnp.zeros((N_TGT, NLP, TVP), jnp.int32)
    tgt = tgt.at[:, :, :NV].set(jnp.transpose(tgtp, (0, 2, 1)))
    align = jnp.concatenate(
        [align_10[:, :10].T,
         jnp.broadcast_to(align_10[0, :10][:, None], (10, NKP - 60))],
        axis=1).astype(jnp.int32)

    out = _lfd_sc(q8_table.reshape(-1), src, tgt, align)
    return jnp.transpose(out[:, :, :, 0], (1, 0, 2)).reshape(N_SRC, N_TGT)


# speedup vs baseline: 1.8194x; 1.0142x over previous
import functools

import jax
import jax.numpy as jnp
from jax import lax
from jax.experimental import pallas as pl
from jax.experimental.pallas import tpu as pltpu
from jax.experimental.pallas import tpu_sc as plsc

N_SRC = 4
N_TGT = 256
NV = 100
NL = 47
NLP = 48
TVP = 112
NKP = 64
I32MAX = 2**31 - 1


def _vperm(x, idx16):
    return lax.gather(
        x, idx16[:, None],
        lax.GatherDimensionNumbers(offset_dims=(), collapsed_slice_dims=(0,),
                                   start_index_map=(0,)),
        (1,), mode=lax.GatherScatterMode.PROMISE_IN_BOUNDS)


def _pack_views(A, F, C, E):
    n = A.shape[0]
    return jnp.concatenate(
        [A.reshape(n, NV, 35), F.reshape(n, NV, 10),
         C.reshape(n, NV, 1), E.reshape(n, NV, 1),
         jnp.zeros((n, NV, 1), jnp.int32)], axis=-1)


def _lfd_sc(q8, src, tgt, align):
    info = plsc.get_sparse_core_info()
    nw = info.num_cores * info.num_subcores
    jpw = tgt.shape[0] // nw
    mesh = plsc.VectorSubcoreMesh(core_axis_name="c", subcore_axis_name="s")

    nj = tgt.shape[0]

    @functools.partial(
        pl.kernel,
        out_type=jax.ShapeDtypeStruct((nw, N_SRC, jpw, 16), jnp.int32),
        mesh=mesh,
        compiler_params=pltpu.CompilerParams(use_tc_tiling_on_sc=False,
                                             needs_layout_passes=False),
        scratch_types=[
            pltpu.VMEM((65536,), jnp.float32),
            pltpu.VMEM((N_SRC, NV, NLP), jnp.int32),
            pltpu.VMEM((NLP, TVP), jnp.int32),
            pltpu.VMEM((10, NKP), jnp.int32),
            pltpu.VMEM((NV * TVP,), jnp.int32),
            pltpu.VMEM((NV * TVP,), jnp.float32),
            pltpu.VMEM((N_SRC, jpw, 16), jnp.int32),
        ],
    )
    def k(q8_hbm, src_hbm, tgt_hbm, align_hbm, out_hbm,
          q_v, src_v, tgt_v, align_v, cost_v, part_v, res_v):
        wid = lax.axis_index("s") * info.num_cores + lax.axis_index("c")
        pltpu.sync_copy(q8_hbm, q_v)
        pltpu.sync_copy(src_hbm, src_v)
        pltpu.sync_copy(align_hbm, align_v)

        lane_sel = [jnp.full((16,), m, jnp.int32) for m in range(16)]

        def per_j(jloc, _):
            pltpu.sync_copy(tgt_hbm.at[wid * jpw + jloc], tgt_v)

            def per_i(i, _):
                def per_tb(tb, _):
                    col = pl.ds(tb * 16, 16)
                    for c in range(3):
                        ls = range(16 * c, min(16 * c + 16, NL))
                        t_vec = {l: tgt_v[l, col] for l in ls}

                        @plsc.parallel_loop(0, NV, 1, unroll=2)
                        def per_sv(sv, c=c, ls=ls, t_vec=t_vec):
                            sa = src_v[i, sv, pl.ds(c * 16, 16)]
                            a1 = [jnp.zeros((16,), jnp.float32)
                                  for _ in range(2)]
                            a2 = [jnp.zeros((16,), jnp.float32)
                                  for _ in range(2)]
                            n1 = n2 = 0
                            for l in ls:
                                row = _vperm(sa, lane_sel[l % 16])
                                g = plsc.load_gather(q_v, [row + t_vec[l]])
                                if 35 <= l <= 45:
                                    a2[n2 % 2] = a2[n2 % 2] + g
                                    n2 += 1
                                else:
                                    a1[n1 % 2] = a1[n1 % 2] + g
                                    n1 += 1
                            w = (a1[0] + a1[1]) + 2.0 * (a2[0] + a2[1])
                            sl = pl.ds(sv * TVP + tb * 16, 16)
                            if c == 0:
                                part_v[sl] = w
                            elif c == 1:
                                part_v[sl] = part_v[sl] + w
                            else:
                                cost_v[sl] = (
                                    part_v[sl] + w).astype(jnp.int32)
                    return 0

                lax.fori_loop(0, TVP // 16, per_tb, 0)

                minv = jnp.full((16,), I32MAX, jnp.int32)
                for kb in range(NKP // 16):
                    a_vec = [align_v[dd, pl.ds(kb * 16, 16)]
                             for dd in range(10)]

                    @plsc.parallel_loop(0, NV, 1, unroll=2, carry=minv)
                    def per_st(st, mv):
                        s1 = st // 10
                        t1 = st % 10
                        pa = [jnp.zeros((16,), jnp.int32) for _ in range(2)]
                        for dd in range(10):
                            base = jnp.full(
                                (16,), (s1 * 10 + dd) * TVP + t1 * 10,
                                jnp.int32)
                            pa[dd % 2] = pa[dd % 2] + plsc.load_gather(
                                cost_v, [base + a_vec[dd]])
                        return jnp.minimum(mv, pa[0] + pa[1])

                    minv = per_st

                m = lax.reduce_min(minv, (0,))
                res_v[i, jloc, :] = jnp.full((16,), m, jnp.int32)
                return 0

            lax.fori_loop(0, N_SRC, per_i, 0)
            return 0

        lax.fori_loop(0, jpw, per_j, 0)
        pltpu.sync_copy(res_v, out_hbm.at[wid])

    out = k(q8, src, tgt, align)
    return jnp.transpose(out[:, :, :, 0], (1, 0, 2)).reshape(N_SRC, nj)


def kernel(q8_table, align_10, src_ArtCoeff, src_FdCoeff_q8, src_CirCoeff_q8,
           src_EccCoeff_q8, tgt_ArtCoeff, tgt_FdCoeff_q8, tgt_CirCoeff_q8,
           tgt_EccCoeff_q8):
    src = _pack_views(src_ArtCoeff, src_FdCoeff_q8,
                      src_CirCoeff_q8, src_EccCoeff_q8) * 256
    tgtp = _pack_views(tgt_ArtCoeff, tgt_FdCoeff_q8,
                       tgt_CirCoeff_q8, tgt_EccCoeff_q8)
    tgt = jnp.zeros((N_TGT, NLP, TVP), jnp.int32)
    tgt = tgt.at[:, :, :NV].set(jnp.transpose(tgtp, (0, 2, 1)))
    align = jnp.concatenate(
        [align_10[:, :10].T,
         jnp.broadcast_to(align_10[0, :10][:, None], (10, NKP - 60))],
        axis=1).astype(jnp.int32)

    q8f = q8_table.reshape(-1)

    ndev = jax.device_count()
    nshard = 1
    for d in (8, 4, 2):
        if ndev >= d and N_TGT % (32 * d) == 0:
            nshard = d
            break
    if nshard > 1:
        import numpy as _np
        from jax.sharding import Mesh, PartitionSpec as P
        from jax.experimental.shard_map import shard_map
        mesh = Mesh(_np.array(jax.devices()[:nshard]), ("x",))
        fn = shard_map(_lfd_sc, mesh=mesh,
                       in_specs=(P(), P(), P("x"), P()),
                       out_specs=P(None, "x"), check_rep=False)
        return fn(q8f, src, tgt, align)
    return _lfd_sc(q8f, src, tgt, align)
